# baseline (speedup 1.0000x reference)
import functools

import jax
import jax.numpy as jnp
from jax import lax
from jax.experimental import pallas as pl
from jax.experimental.pallas import tpu as pltpu
from jax.experimental.pallas import tpu_sc as plsc

D = 128
_NC, _NS, _NL = 2, 16, 16
_NW = _NC * _NS
_CH = 64
_NSLOT = 4


def _mm_body(x_ref, w1_ref, w2_ref, b_ref, t1_ref, t2_ref):
    xb = x_ref[...]
    t1 = jnp.dot(xb, w1_ref[...], preferred_element_type=jnp.float32)
    t1_ref[...] = t1 + b_ref[0:1, :]
    t2_ref[...] = jnp.dot(xb, w2_ref[...], preferred_element_type=jnp.float32)


def _node_tables(x, W, b):
    n, d = x.shape
    blk = 1000
    w1 = W[:d]
    w2 = W[d:]
    bp = jnp.tile(b.reshape(1, D), (8, 1))
    t1, t2 = pl.pallas_call(
        _mm_body,
        grid=(n // blk,),
        in_specs=[
            pl.BlockSpec((blk, d), lambda i: (i, 0)),
            pl.BlockSpec((d, D), lambda i: (0, 0)),
            pl.BlockSpec((d, D), lambda i: (0, 0)),
            pl.BlockSpec((8, D), lambda i: (0, 0)),
        ],
        out_specs=[
            pl.BlockSpec((blk, D), lambda i: (i, 0)),
            pl.BlockSpec((blk, D), lambda i: (i, 0)),
        ],
        out_shape=[
            jax.ShapeDtypeStruct((n, D), jnp.float32),
            jax.ShapeDtypeStruct((n, D), jnp.float32),
        ],
    )(x, w1, w2, bp)
    return t1, t2


def _edge_apply(t1, t2, ei_flat):
    E = ei_flat.shape[0] // 2
    epw = E // _NW
    nchunk = epw // _CH
    tail = epw - nchunk * _CH
    assert nchunk % _NSLOT == 0 and nchunk // _NSLOT >= 2 and tail % 8 == 0
    ngroup = nchunk // _NSLOT
    mesh = plsc.VectorSubcoreMesh(
        core_axis_name="c", subcore_axis_name="s",
        num_cores=_NC, num_subcores=_NS,
    )

    buf = pltpu.VMEM((_CH, D), jnp.float32)

    @functools.partial(
        pl.kernel,
        out_type=jax.ShapeDtypeStruct((E, D), jnp.float32),
        mesh=mesh,
        scratch_types=[
            pltpu.VMEM((epw,), jnp.int32),
            pltpu.VMEM((epw,), jnp.int32),
        ] + [buf] * (3 * _NSLOT) + [pltpu.SemaphoreType.DMA] * (3 * _NSLOT),
    )
    def k(t1_hbm, t2_hbm, ei_hbm, out_hbm,
          srcv, dstv,
          ba0, bb0, bo0, ba1, bb1, bo1, ba2, bb2, bo2, ba3, bb3, bo3,
          sga0, sgb0, so0, sga1, sgb1, so1, sga2, sgb2, so2,
          sga3, sgb3, so3):
        wid = lax.axis_index("s") * _NC + lax.axis_index("c")
        ebase = wid * epw
        pltpu.sync_copy(ei_hbm.at[pl.ds(ebase, epw)], srcv)
        pltpu.sync_copy(ei_hbm.at[pl.ds(E + ebase, epw)], dstv)

        bufs = ((ba0, bb0, bo0, sga0, sgb0, so0),
                (ba1, bb1, bo1, sga1, sgb1, so1),
                (ba2, bb2, bo2, sga2, sgb2, so2),
                (ba3, bb3, bo3, sga3, sgb3, so3))

        def gathers(c, s):
            ba, bb, _, sga, sgb, _ = bufs[s]
            off = c * _CH
            cpa = pltpu.make_async_copy(
                t1_hbm.at[srcv.at[pl.ds(off, _CH)]], ba, sga)
            cpb = pltpu.make_async_copy(
                t2_hbm.at[dstv.at[pl.ds(off, _CH)]], bb, sgb)
            return cpa, cpb

        def store_cp(c, s):
            _, _, bo, _, _, so = bufs[s]
            return pltpu.make_async_copy(
                bo, out_hbm.at[pl.ds(ebase + c * _CH, _CH)], so)

        def fire(c, s):
            cpa, cpb = gathers(c, s)
            cpa.start()
            cpb.start()

        def wait_gathers(c, s):
            cpa, cpb = gathers(c, s)
            cpa.wait()
            cpb.wait()

        def compute(s):
            ba, bb, bo, *_ = bufs[s]

            @plsc.parallel_loop(0, _CH, step=1, unroll=4)
            def _row(r):
                for j in range(D // _NL):
                    sl = pl.ds(j * _NL, _NL)
                    bo[r, sl] = jnp.maximum(ba[r, sl] + bb[r, sl], 0.0)

        for s in range(_NSLOT):
            fire(s, s)

        def grp(q, carry):
            for s in range(_NSLOT):
                c = _NSLOT * q + s
                wait_gathers(c, s)

                @pl.when(q > 0)
                def _():
                    store_cp(c - _NSLOT, s).wait()

                compute(s)
                store_cp(c, s).start()
                fire(c + _NSLOT, s)
            return carry

        lax.fori_loop(0, ngroup - 1, grp, 0)

        if tail:
            toff = nchunk * _CH
            tail_ga = pltpu.make_async_copy(
                t1_hbm.at[srcv.at[pl.ds(toff, tail)]],
                ba0.at[pl.ds(0, tail)], sga0)
            tail_gb = pltpu.make_async_copy(
                t2_hbm.at[dstv.at[pl.ds(toff, tail)]],
                bb0.at[pl.ds(0, tail)], sgb0)
            tail_st = pltpu.make_async_copy(
                bo0.at[pl.ds(0, tail)],
                out_hbm.at[pl.ds(ebase + toff, tail)], so0)

        for s in range(_NSLOT):
            c = nchunk - _NSLOT + s
            wait_gathers(c, s)
            store_cp(c - _NSLOT, s).wait()
            compute(s)
            store_cp(c, s).start()
            if s == 0 and tail:
                tail_ga.start()
                tail_gb.start()

        if tail:
            tail_ga.wait()
            tail_gb.wait()
            store_cp(nchunk - _NSLOT, 0).wait()

            @plsc.parallel_loop(0, tail, step=1, unroll=4)
            def _trow(r):
                for j in range(D // _NL):
                    sl = pl.ds(j * _NL, _NL)
                    bo0[r, sl] = jnp.maximum(ba0[r, sl] + bb0[r, sl], 0.0)

            tail_st.start()
        else:
            store_cp(nchunk - _NSLOT, 0).wait()
        for s in range(1, _NSLOT):
            store_cp(nchunk - _NSLOT + s, s).wait()
        if tail:
            tail_st.wait()

    return k(t1, t2, ei_flat)


def kernel(x, edge_index, W, b):
    t1, t2 = _node_tables(x, W, b)
    return _edge_apply(t1, t2, edge_index.reshape(-1))


# speedup vs baseline: 1.0138x; 1.0138x over previous
import functools

import jax
import jax.numpy as jnp
from jax import lax
from jax.experimental import pallas as pl
from jax.experimental.pallas import tpu as pltpu
from jax.experimental.pallas import tpu_sc as plsc

D = 128
_NC, _NS, _NL = 2, 16, 16
_NW = _NC * _NS
_CH = 80


def _mm_body(x_ref, w1_ref, w2_ref, b_ref, t1_ref, t2_ref):
    xb = x_ref[...]
    t1 = jnp.dot(xb, w1_ref[...], preferred_element_type=jnp.float32)
    t1_ref[...] = t1 + b_ref[0:1, :]
    t2_ref[...] = jnp.dot(xb, w2_ref[...], preferred_element_type=jnp.float32)


def _node_tables(x, W, b):
    n, d = x.shape
    blk = 1000
    w1 = W[:d]
    w2 = W[d:]
    bp = jnp.tile(b.reshape(1, D), (8, 1))
    t1, t2 = pl.pallas_call(
        _mm_body,
        grid=(n // blk,),
        in_specs=[
            pl.BlockSpec((blk, d), lambda i: (i, 0)),
            pl.BlockSpec((d, D), lambda i: (0, 0)),
            pl.BlockSpec((d, D), lambda i: (0, 0)),
            pl.BlockSpec((8, D), lambda i: (0, 0)),
        ],
        out_specs=[
            pl.BlockSpec((blk, D), lambda i: (i, 0)),
            pl.BlockSpec((blk, D), lambda i: (i, 0)),
        ],
        out_shape=[
            jax.ShapeDtypeStruct((n, D), jnp.float32),
            jax.ShapeDtypeStruct((n, D), jnp.float32),
        ],
    )(x, w1, w2, bp)
    return t1, t2


def _edge_apply(t1, t2, ei_flat):
    E = ei_flat.shape[0] // 2
    epw = E // _NW
    nchunk = epw // _CH
    tail = epw - nchunk * _CH
    assert nchunk % 3 == 2 and nchunk >= 5 and tail % 8 == 0
    ntri = (nchunk - 2) // 3
    mesh = plsc.VectorSubcoreMesh(
        core_axis_name="c", subcore_axis_name="s",
        num_cores=_NC, num_subcores=_NS,
    )

    gbuf = pltpu.VMEM((_CH, D), jnp.float32)
    obuf = pltpu.VMEM((_CH, D), jnp.float32)

    @functools.partial(
        pl.kernel,
        out_type=jax.ShapeDtypeStruct((E, D), jnp.float32),
        mesh=mesh,
        scratch_types=[
            pltpu.VMEM((epw,), jnp.int32),
            pltpu.VMEM((epw,), jnp.int32),
            gbuf, gbuf, obuf, gbuf, gbuf, obuf, gbuf, gbuf, obuf,
        ] + [pltpu.SemaphoreType.DMA] * 9,
    )
    def k(t1_hbm, t2_hbm, ei_hbm, out_hbm,
          srcv, dstv, ba0, bb0, bo0, ba1, bb1, bo1, ba2, bb2, bo2,
          sga0, sgb0, so0, sga1, sgb1, so1, sga2, sgb2, so2):
        wid = lax.axis_index("s") * _NC + lax.axis_index("c")
        ebase = wid * epw
        pltpu.sync_copy(ei_hbm.at[pl.ds(ebase, epw)], srcv)
        pltpu.sync_copy(ei_hbm.at[pl.ds(E + ebase, epw)], dstv)

        bufs = ((ba0, bb0, bo0, sga0, sgb0, so0),
                (ba1, bb1, bo1, sga1, sgb1, so1),
                (ba2, bb2, bo2, sga2, sgb2, so2))

        def gathers(c, s):
            ba, bb, _, sga, sgb, _ = bufs[s]
            off = c * _CH
            cpa = pltpu.make_async_copy(
                t1_hbm.at[srcv.at[pl.ds(off, _CH)]], ba, sga)
            cpb = pltpu.make_async_copy(
                t2_hbm.at[dstv.at[pl.ds(off, _CH)]], bb, sgb)
            return cpa, cpb

        def store_cp(c, s):
            _, _, bo, _, _, so = bufs[s]
            return pltpu.make_async_copy(
                bo, out_hbm.at[pl.ds(ebase + c * _CH, _CH)], so)

        def fire(c, s):
            cpa, cpb = gathers(c, s)
            cpa.start()
            cpb.start()

        def wait_gathers(c, s):
            cpa, cpb = gathers(c, s)
            cpa.wait()
            cpb.wait()

        def compute(s):
            ba, bb, bo, *_ = bufs[s]

            @plsc.parallel_loop(0, _CH, step=1, unroll=4)
            def _row(r):
                for j in range(D // _NL):
                    sl = pl.ds(j * _NL, _NL)
                    bo[r, sl] = jnp.maximum(ba[r, sl] + bb[r, sl], 0.0)

        fire(0, 0)
        fire(1, 1)
        fire(2, 2)

        def tri(q, carry):
            for s in range(3):
                c = 3 * q + s
                wait_gathers(c, s)

                @pl.when(q > 0)
                def _():
                    store_cp(c - 3, s).wait()

                compute(s)
                store_cp(c, s).start()
                if s < 2:
                    fire(c + 3, s)
                else:
                    @pl.when(q < ntri - 1)
                    def _():
                        fire(c + 3, s)
            return carry

        lax.fori_loop(0, ntri, tri, 0)

        if tail:
            toff = nchunk * _CH
            tail_ga = pltpu.make_async_copy(
                t1_hbm.at[srcv.at[pl.ds(toff, tail)]],
                ba2.at[pl.ds(0, tail)], sga2)
            tail_gb = pltpu.make_async_copy(
                t2_hbm.at[dstv.at[pl.ds(toff, tail)]],
                bb2.at[pl.ds(0, tail)], sgb2)
            tail_st = pltpu.make_async_copy(
                bo2.at[pl.ds(0, tail)],
                out_hbm.at[pl.ds(ebase + toff, tail)], so2)
            tail_ga.start()
            tail_gb.start()

        for s, c in ((0, nchunk - 2), (1, nchunk - 1)):
            wait_gathers(c, s)
            store_cp(c - 3, s).wait()
            compute(s)
            store_cp(c, s).start()

        if tail:
            tail_ga.wait()
            tail_gb.wait()
            store_cp(nchunk - 3, 2).wait()

            @plsc.parallel_loop(0, tail, step=1, unroll=4)
            def _trow(r):
                for j in range(D // _NL):
                    sl = pl.ds(j * _NL, _NL)
                    bo2[r, sl] = jnp.maximum(ba2[r, sl] + bb2[r, sl], 0.0)

            tail_st.start()
        else:
            store_cp(nchunk - 3, 2).wait()
        store_cp(nchunk - 2, 0).wait()
        store_cp(nchunk - 1, 1).wait()
        if tail:
            tail_st.wait()

    return k(t1, t2, ei_flat)


def kernel(x, edge_index, W, b):
    t1, t2 = _node_tables(x, W, b)
    return _edge_apply(t1, t2, edge_index.reshape(-1))
